# baseline (speedup 1.0000x reference)
import functools

import jax
import jax.numpy as jnp
from jax import lax
from jax.experimental import pallas as pl
from jax.experimental.pallas import tpu as pltpu
from jax.experimental.pallas import tpu_sc as plsc

T, D, E, K = 2048, 768, 8, 2
H = 4 * D
A = T * K
NB = 256
G_MAX = A // NB + E
A_PAD = G_MAX * NB

NC, NS, L = 2, 16, 16
NW = NC * NS


@functools.cache
def _mesh():
    return plsc.VectorSubcoreMesh(core_axis_name="c", subcore_axis_name="s")


_SC_PARAMS = pltpu.CompilerParams(needs_layout_passes=False)


def _router_body(x_ref, n_ref, wg_ref, bg_ref, wn_ref, bn_ref,
                 e0_ref, e1_ref, g0_ref, g1_ref):
    xb = x_ref[...]
    logits = jnp.dot(xb, wg_ref[...],
                     preferred_element_type=jnp.float32) + bg_ref[...]
    nlog = jnp.dot(xb, wn_ref[...],
                   preferred_element_type=jnp.float32) + bn_ref[...]
    sp = jnp.maximum(nlog, 0.0) + jnp.log1p(jnp.exp(-jnp.abs(nlog)))
    noisy = logits + n_ref[...] * sp
    lane = lax.broadcasted_iota(jnp.int32, (T, E), 1)
    top1 = jnp.max(noisy, axis=1, keepdims=True)
    idx1 = jnp.min(jnp.where(noisy == top1, lane, E), axis=1, keepdims=True)
    noisy2 = jnp.where(lane == idx1, -jnp.inf, noisy)
    top2 = jnp.max(noisy2, axis=1, keepdims=True)
    idx2 = jnp.min(jnp.where(noisy2 == top2, lane, E), axis=1, keepdims=True)
    sel = (lane == idx1) | (lane == idx2)
    p = jnp.where(sel, jnp.exp(noisy - top1), 0.0)
    z = jnp.sum(p, axis=1, keepdims=True)
    p1 = jnp.sum(jnp.where(lane == idx1, p, 0.0), axis=1, keepdims=True)
    p2 = jnp.sum(jnp.where(lane == idx2, p, 0.0), axis=1, keepdims=True)
    e0_ref[...] = idx1
    e1_ref[...] = idx2
    g0_ref[...] = p1 / z
    g1_ref[...] = p2 / z


def _router(xf, nf, Wg, bg, Wn, bn):
    full = lambda s: pl.BlockSpec(s, lambda: (0,) * len(s))
    return pl.pallas_call(
        _router_body,
        in_specs=[full((T, D)), full((T, E)), full((D, E)), full((1, E)),
                  full((D, E)), full((1, E))],
        out_specs=[full((T, 1)), full((T, 1)), full((T, 1)), full((T, 1))],
        out_shape=[jax.ShapeDtypeStruct((T, 1), jnp.int32),
                   jax.ShapeDtypeStruct((T, 1), jnp.int32),
                   jax.ShapeDtypeStruct((T, 1), jnp.float32),
                   jax.ShapeDtypeStruct((T, 1), jnp.float32)],
    )(xf, nf, Wg, bg.reshape(1, E), Wn, bn.reshape(1, E))


def _sort_body(e0_hbm, e1_hbm, g0_hbm, g1_hbm,
               tok_hbm, gate_hbm, bexp_hbm, dest_hbm,
               ev, gv, rank, destv, toks, gts, offs_v, bexp_v):
    wid = lax.axis_index("s") * NC + lax.axis_index("c")

    @pl.when(wid == 0)
    def _():
        pltpu.sync_copy(e0_hbm, ev.at[pl.ds(0, T)])
        pltpu.sync_copy(e1_hbm, ev.at[pl.ds(T, T)])
        pltpu.sync_copy(g0_hbm, gv.at[pl.ds(0, T)])
        pltpu.sync_copy(g1_hbm, gv.at[pl.ds(T, T)])

        def p1(i, cnts):
            v = ev[pl.ds(i * L, L)]
            r = jnp.zeros((L,), jnp.int32)
            one = jnp.ones((L,), jnp.int32)
            new = []
            for ex in range(E):
                m = v == jnp.full((L,), ex, jnp.int32)
                mi = jnp.where(m, one, jnp.zeros((L,), jnp.int32))
                pc = plsc.cumsum(mi)
                r = jnp.where(m, cnts[ex] + pc - one, r)
                tot = jnp.full((L,), jnp.sum(mi), jnp.int32)
                new.append(cnts[ex] + tot)
            rank[pl.ds(i * L, L)] = r
            return tuple(new)
        zz = jnp.zeros((L,), jnp.int32)
        counts = lax.fori_loop(0, A // L, p1, (zz,) * E)

        iota = lax.iota(jnp.int32, L)
        nbv = jnp.full((L,), NB, jnp.int32)
        nbm1 = jnp.full((L,), NB - 1, jnp.int32)
        cb = jnp.zeros((L,), jnp.int32)
        off_v = jnp.zeros((L,), jnp.int32)
        be0 = jnp.zeros((L,), jnp.int32)
        be1 = jnp.zeros((L,), jnp.int32)
        one = jnp.ones((L,), jnp.int32)
        iota_hi = iota + jnp.full((L,), L, jnp.int32)
        for ex in range(E):
            off_v = jnp.where(iota == jnp.full((L,), ex, jnp.int32),
                              cb * nbv, off_v)
            cb = cb + (counts[ex] + nbm1) // nbv
            be0 = be0 + jnp.where(iota >= cb, one, zz)
            be1 = be1 + jnp.where(iota_hi >= cb, one, zz)
        emax = jnp.full((L,), E - 1, jnp.int32)
        offs_v[pl.ds(0, L)] = off_v
        bexp_v[pl.ds(0, L)] = jnp.minimum(be0, emax)
        bexp_v[pl.ds(L, L)] = jnp.minimum(be1, emax)
        pltpu.sync_copy(bexp_v, bexp_hbm)

        def pz(i, _):
            toks[pl.ds(i * L, L)] = jnp.zeros((L,), jnp.int32)
            gts[pl.ds(i * L, L)] = jnp.zeros((L,), jnp.float32)
            return 0
        lax.fori_loop(0, A_PAD // L, pz, 0)

        def p2(i, _):
            v = ev[pl.ds(i * L, L)]
            r = rank[pl.ds(i * L, L)]
            g = gv[pl.ds(i * L, L)]
            base = plsc.load_gather(offs_v, [v])
            dst = base + r
            tok = (jnp.full((L,), i * L, jnp.int32) + iota) & jnp.full(
                (L,), T - 1, jnp.int32)
            plsc.store_scatter(toks, [dst], tok)
            plsc.store_scatter(gts, [dst], g)
            destv[pl.ds(i * L, L)] = dst
            return 0
        lax.fori_loop(0, A // L, p2, 0)

        pltpu.sync_copy(toks, tok_hbm)
        pltpu.sync_copy(gts, gate_hbm)
        pltpu.sync_copy(destv, dest_hbm)


def _sort(*args):
    return pl.kernel(
        _sort_body,
        mesh=_mesh(),
        compiler_params=_SC_PARAMS,
        out_type=[jax.ShapeDtypeStruct((A_PAD,), jnp.int32),
                  jax.ShapeDtypeStruct((A_PAD,), jnp.float32),
                  jax.ShapeDtypeStruct((2 * L,), jnp.int32),
                  jax.ShapeDtypeStruct((A,), jnp.int32)],
        scratch_types=[pltpu.VMEM((A,), jnp.int32),
                       pltpu.VMEM((A,), jnp.float32),
                       pltpu.VMEM((A,), jnp.int32),
                       pltpu.VMEM((A,), jnp.int32),
                       pltpu.VMEM((A_PAD,), jnp.int32),
                       pltpu.VMEM((A_PAD,), jnp.float32),
                       pltpu.VMEM((L,), jnp.int32),
                       pltpu.VMEM((2 * L,), jnp.int32)],
    )(*args)


_G_CHUNK = A_PAD // NW // 2


def _gather_body(x_hbm, tok_hbm, xg_hbm, idx_v, rows_v, sem):
    wid = lax.axis_index("s") * NC + lax.axis_index("c")
    for c in range(2):
        base = wid * (2 * _G_CHUNK) + c * _G_CHUNK
        pltpu.sync_copy(tok_hbm.at[pl.ds(base, _G_CHUNK)], idx_v)
        pltpu.async_copy(x_hbm.at[idx_v], rows_v, sem).wait()
        pltpu.sync_copy(rows_v, xg_hbm.at[pl.ds(base, _G_CHUNK)])


def _gather(*args):
    return pl.kernel(
        _gather_body,
        mesh=_mesh(),
        compiler_params=_SC_PARAMS,
        out_type=jax.ShapeDtypeStruct((A_PAD, D), jnp.float32),
        scratch_types=[pltpu.VMEM((_G_CHUNK,), jnp.int32),
                       pltpu.VMEM((_G_CHUNK, D), jnp.float32),
                       pltpu.SemaphoreType.DMA],
    )(*args)


def _ffn_body(bexp_ref, xg_ref, gate_ref, w1_ref, b1_ref, w2_ref, b2_ref,
              out_ref):
    xb = xg_ref[...].astype(jnp.bfloat16)
    w1 = w1_ref[0].astype(jnp.bfloat16)
    h = jnp.dot(xb, w1, preferred_element_type=jnp.float32) + b1_ref[0]
    h = jnp.maximum(h, 0.0).astype(jnp.bfloat16)
    w2 = w2_ref[0].astype(jnp.bfloat16)
    o = jnp.dot(h, w2, preferred_element_type=jnp.float32) + b2_ref[0]
    out_ref[...] = o * gate_ref[...]


def _ffn(bexp, xg, gates, W1, b1, W2, b2):
    return pl.pallas_call(
        _ffn_body,
        grid_spec=pltpu.PrefetchScalarGridSpec(
            num_scalar_prefetch=1,
            grid=(G_MAX,),
            in_specs=[
                pl.BlockSpec((NB, D), lambda g, be: (g, 0)),
                pl.BlockSpec((NB, 1), lambda g, be: (g, 0)),
                pl.BlockSpec((1, D, H), lambda g, be: (be[g], 0, 0)),
                pl.BlockSpec((1, 1, H), lambda g, be: (be[g], 0, 0)),
                pl.BlockSpec((1, H, D), lambda g, be: (be[g], 0, 0)),
                pl.BlockSpec((1, 1, D), lambda g, be: (be[g], 0, 0)),
            ],
            out_specs=pl.BlockSpec((NB, D), lambda g, be: (g, 0)),
        ),
        out_shape=jax.ShapeDtypeStruct((A_PAD, D), jnp.float32),
        compiler_params=pltpu.CompilerParams(
            vmem_limit_bytes=100 * 1024 * 1024),
    )(bexp, xg, gates, W1, b1.reshape(E, 1, H), W2, b2.reshape(E, 1, D))


_C_CHUNK = T // NW


def _combine_body(os_hbm, dest_hbm, fin_hbm, d_v, acc_v, row_v, sem):
    wid = lax.axis_index("s") * NC + lax.axis_index("c")
    base = wid * _C_CHUNK
    pltpu.sync_copy(dest_hbm.at[pl.ds(base, _C_CHUNK)], d_v)
    pltpu.async_copy(os_hbm.at[d_v], acc_v, sem).wait()
    pltpu.sync_copy(dest_hbm.at[pl.ds(T + base, _C_CHUNK)], d_v)
    pltpu.async_copy(os_hbm.at[d_v], row_v, sem).wait()

    def add(i, _):
        for j in range(D // L):
            s = pl.ds(j * L, L)
            acc_v[i, s] = acc_v[i, s] + row_v[i, s]
        return 0
    lax.fori_loop(0, _C_CHUNK, add, 0)
    pltpu.sync_copy(acc_v, fin_hbm.at[pl.ds(base, _C_CHUNK)])


def _combine(*args):
    return pl.kernel(
        _combine_body,
        mesh=_mesh(),
        compiler_params=_SC_PARAMS,
        out_type=jax.ShapeDtypeStruct((T, D), jnp.float32),
        scratch_types=[pltpu.VMEM((_C_CHUNK,), jnp.int32),
                       pltpu.VMEM((_C_CHUNK, D), jnp.float32),
                       pltpu.VMEM((_C_CHUNK, D), jnp.float32),
                       pltpu.SemaphoreType.DMA],
    )(*args)


@jax.jit
def kernel(x, noise, Wg, bg, Wn, bn, W1, b1, W2, b2):
    xf = x.reshape(T, D)
    nf = noise.reshape(T, E)
    e0, e1, g0, g1 = _router(xf, nf, Wg, bg, Wn, bn)
    tok_sorted, gate_sorted, bexp, dest = _sort(
        e0.reshape(T), e1.reshape(T), g0.reshape(T), g1.reshape(T))
    xg = _gather(xf, tok_sorted)
    out_sorted = _ffn(bexp[:G_MAX], xg, gate_sorted.reshape(A_PAD, 1),
                      W1, b1, W2, b2)
    final = _combine(out_sorted, dest)
    return final.reshape(1, T, D)
